# baseline (speedup 1.0000x reference)
import functools

import jax
import jax.numpy as jnp
from jax import lax
from jax.experimental import pallas as pl
from jax.experimental.pallas import tpu as pltpu
from jax.experimental.pallas import tpu_sc as plsc

NW = 32
NS = 16
LANES = 16
CH = 80


def _sc_degrees(src_hbm, dst_hbm, degp_hbm, idxv, hist, redv, segv, shared):
    np_, = hist.shape
    seg = np_ // NS
    c = lax.axis_index("c")
    s = lax.axis_index("s")
    w = c * NS + s
    nch, ch = idxv.shape
    zero16 = jnp.zeros((LANES,), jnp.float32)
    one16 = jnp.full((LANES,), 1.0, jnp.float32)
    for direction, eb in enumerate((src_hbm, dst_hbm)):
        @pl.loop(0, np_ // LANES)
        def _(i):
            hist[pl.ds(i * LANES, LANES)] = zero16

        pltpu.sync_copy(eb.at[w], idxv)

        @pl.loop(0, nch)
        def _(j):
            for k in range(ch // LANES):
                ids = idxv[j, pl.ds(k * LANES, LANES)]
                plsc.addupdate_scatter(hist, [ids], one16)

        pltpu.sync_copy(hist, shared.at[s])
        plsc.subcore_barrier()
        for r in range(NS):
            pltpu.sync_copy(shared.at[r, pl.ds(s * seg, seg)], redv.at[r])

        @pl.loop(0, seg // LANES)
        def _(k):
            acc = redv[0, pl.ds(k * LANES, LANES)]
            for r in range(1, NS):
                acc = acc + redv[r, pl.ds(k * LANES, LANES)]
            segv[pl.ds(k * LANES, LANES)] = acc

        pltpu.sync_copy(segv, degp_hbm.at[direction, c, pl.ds(s * seg, seg)])
        plsc.subcore_barrier()


def _sc_agg(h_hbm, src_hbm, dst_hbm, aggp_hbm, idxs, idxd, rows, shared, sem):
    np_, d = shared.shape
    seg = np_ // NS
    c = lax.axis_index("c")
    s = lax.axis_index("s")
    w = c * NS + s
    nch, ch = idxs.shape
    zero16 = jnp.zeros((LANES,), jnp.float32)

    @pl.loop(0, ch)
    def _(j):
        for k in range(d // LANES):
            rows[j, pl.ds(k * LANES, LANES)] = zero16

    for k in range(seg // ch):
        pltpu.sync_copy(rows, shared.at[pl.ds(s * seg + k * ch, ch)])
    plsc.subcore_barrier()

    pltpu.sync_copy(src_hbm.at[w], idxs)
    pltpu.sync_copy(dst_hbm.at[w], idxd)

    @pl.loop(0, nch)
    def _(j):
        pltpu.async_copy(h_hbm.at[idxs.at[j]], rows, sem).wait()
        pltpu.sync_copy(rows, shared.at[idxd.at[j]], add=True)

    plsc.subcore_barrier()
    pltpu.sync_copy(shared.at[pl.ds(s * seg, seg)],
                    aggp_hbm.at[c, pl.ds(s * seg, seg)])


def _tc_norm_body(n, np_, x_ref, degp_ref, h_ref, nd_ref):
    do = degp_ref[0, 0, :] + degp_ref[0, 1, :]
    di = degp_ref[1, 0, :] + degp_ref[1, 1, :]
    ns_ = lax.rsqrt(jnp.maximum(do, 1.0))
    nd_ref[...] = lax.rsqrt(jnp.maximum(di, 1.0)).reshape(1, np_)
    h_ref[pl.ds(0, n), :] = x_ref[...] * ns_[:n][:, None]


def _tc_main_body(n, s_seq, l_seq, d,
                  aggp_ref, nd_ref, seq_ref, wc_ref, bc_ref, w1_ref, b1_ref,
                  wih_ref, whh_ref, bi_ref, bh_ref, w2_ref, b2_ref,
                  final_ref, outv, xv, xgv, ysv):
    agg = aggp_ref[0, pl.ds(0, n), :] + aggp_ref[1, pl.ds(0, n), :]
    agg = agg * nd_ref[0, :n][:, None]
    med = jnp.dot(agg, wc_ref[...], preferred_element_type=jnp.float32) + bc_ref[...]
    outv[...] = jnp.dot(med, w1_ref[...], preferred_element_type=jnp.float32) + b1_ref[...]
    a_mat = wih_ref[...]
    b_mat = whh_ref[...].astype(jnp.bfloat16)
    bsum = bi_ref[...] + bh_ref[...]
    dn_t = (((1,), (1,)), ((), ()))

    def seq_body(si, _):
        def gather_body(t, _):
            idx = seq_ref[si, t]
            xv[pl.ds(t, 1), :] = outv[pl.ds(idx, 1), :]
            return 0
        lax.fori_loop(0, l_seq, gather_body, 0)
        xgv[...] = lax.dot_general(
            xv[...], a_mat, dn_t, preferred_element_type=jnp.float32) + bsum

        def step(t, hc):
            h, cc = hc
            g = xgv[pl.ds(t, 1), :] + lax.dot_general(
                h.astype(jnp.bfloat16), b_mat, dn_t,
                preferred_element_type=jnp.float32)
            ig = jax.nn.sigmoid(g[:, 0:d])
            fg = jax.nn.sigmoid(g[:, d:2 * d])
            gg = jnp.tanh(g[:, 2 * d:3 * d])
            og = jax.nn.sigmoid(g[:, 3 * d:4 * d])
            cc = fg * cc + ig * gg
            h = og * jnp.tanh(cc)
            ysv[pl.ds(t, 1), :] = h
            return (h, cc)

        zero_h = jnp.zeros((1, d), jnp.float32)
        lax.fori_loop(0, l_seq, step, (zero_h, zero_h))

        def scat_body(t, _):
            idx = seq_ref[si, t]
            outv[pl.ds(idx, 1), :] = ysv[pl.ds(t, 1), :]
            return 0
        lax.fori_loop(0, l_seq, scat_body, 0)
        return 0

    lax.fori_loop(0, s_seq, seq_body, 0)
    final_ref[...] = jnp.dot(outv[...], w2_ref[...],
                             preferred_element_type=jnp.float32) + b2_ref[...]


def kernel(in_feats, edge_index, seq_ids, W_conv, b_conv, W_ff1, b_ff1,
           W_ih, W_hh, b_ih, b_hh, W_ff2, b_ff2):
    n, d = in_feats.shape
    e = edge_index.shape[1]
    s_seq, l_seq = seq_ids.shape
    np_ = ((n + NS * LANES - 1) // (NS * LANES)) * (NS * LANES)
    epw = e // NW
    nch = epw // CH

    src3 = edge_index[0].reshape(NW, nch, CH)
    dst3 = edge_index[1].reshape(NW, nch, CH)

    mesh = plsc.VectorSubcoreMesh(core_axis_name="c", subcore_axis_name="s")
    seg = np_ // NS

    degp = pl.kernel(
        _sc_degrees,
        out_type=jax.ShapeDtypeStruct((2, 2, np_), jnp.float32),
        mesh=mesh,
        compiler_params=pltpu.CompilerParams(needs_layout_passes=False),
        scratch_types=[
            pltpu.VMEM((nch, CH), jnp.int32),
            pltpu.VMEM((np_,), jnp.float32),
            pltpu.VMEM((NS, seg), jnp.float32),
            pltpu.VMEM((seg,), jnp.float32),
            pltpu.VMEM_SHARED((NS, np_), jnp.float32),
        ],
    )(src3, dst3)

    h, norm_dst = pl.pallas_call(
        functools.partial(_tc_norm_body, n, np_),
        out_shape=(
            jax.ShapeDtypeStruct((np_, d), jnp.float32),
            jax.ShapeDtypeStruct((1, np_), jnp.float32),
        ),
    )(in_feats, degp)

    aggp = pl.kernel(
        _sc_agg,
        out_type=jax.ShapeDtypeStruct((2, np_, d), jnp.float32),
        mesh=mesh,
        scratch_types=[
            pltpu.VMEM((nch, CH), jnp.int32),
            pltpu.VMEM((nch, CH), jnp.int32),
            pltpu.VMEM((CH, d), jnp.float32),
            pltpu.VMEM_SHARED((np_, d), jnp.float32),
            pltpu.SemaphoreType.DMA,
        ],
    )(h, src3, dst3)

    final = pl.pallas_call(
        functools.partial(_tc_main_body, n, s_seq, l_seq, d),
        out_shape=jax.ShapeDtypeStruct((n, d), jnp.float32),
        in_specs=[
            pl.BlockSpec(memory_space=pltpu.VMEM),
            pl.BlockSpec(memory_space=pltpu.VMEM),
            pl.BlockSpec(memory_space=pltpu.SMEM),
        ] + [pl.BlockSpec(memory_space=pltpu.VMEM)] * 10,
        out_specs=pl.BlockSpec(memory_space=pltpu.VMEM),
        scratch_shapes=[
            pltpu.VMEM((n, d), jnp.float32),
            pltpu.VMEM((l_seq, d), jnp.float32),
            pltpu.VMEM((l_seq, 4 * d), jnp.float32),
            pltpu.VMEM((l_seq, d), jnp.float32),
        ],
    )(aggp, norm_dst, seq_ids,
      W_conv, b_conv.reshape(1, d), W_ff1, b_ff1.reshape(1, d),
      W_ih, W_hh, b_ih.reshape(1, 4 * d), b_hh.reshape(1, 4 * d),
      W_ff2, b_ff2.reshape(1, d))
    return final


# speedup vs baseline: 1.2759x; 1.2759x over previous
import functools

import jax
import jax.numpy as jnp
from jax import lax
from jax.experimental import pallas as pl
from jax.experimental.pallas import tpu as pltpu
from jax.experimental.pallas import tpu_sc as plsc

NW = 32
NS = 16
LANES = 16
CH = 80


def _sc_degrees(src_hbm, dst_hbm, degp_hbm, idxv, hist, redv, segv, shared):
    np_, = hist.shape
    seg = np_ // NS
    c = lax.axis_index("c")
    s = lax.axis_index("s")
    w = c * NS + s
    nch, ch = idxv.shape
    zero16 = jnp.zeros((LANES,), jnp.float32)
    one16 = jnp.full((LANES,), 1.0, jnp.float32)
    for direction, eb in enumerate((src_hbm, dst_hbm)):
        @pl.loop(0, np_ // LANES)
        def _(i):
            hist[pl.ds(i * LANES, LANES)] = zero16

        pltpu.sync_copy(eb.at[w], idxv)

        @pl.loop(0, nch)
        def _(j):
            for k in range(ch // LANES):
                ids = idxv[j, pl.ds(k * LANES, LANES)]
                plsc.addupdate_scatter(hist, [ids], one16)

        pltpu.sync_copy(hist, shared.at[s])
        plsc.subcore_barrier()
        for r in range(NS):
            pltpu.sync_copy(shared.at[r, pl.ds(s * seg, seg)], redv.at[r])

        @pl.loop(0, seg // LANES)
        def _(k):
            acc = redv[0, pl.ds(k * LANES, LANES)]
            for r in range(1, NS):
                acc = acc + redv[r, pl.ds(k * LANES, LANES)]
            segv[pl.ds(k * LANES, LANES)] = acc

        pltpu.sync_copy(segv, degp_hbm.at[direction, c, pl.ds(s * seg, seg)])
        plsc.subcore_barrier()


def _sc_agg(h_hbm, src_hbm, dst_hbm, aggp_hbm, idxs, idxd, rows, shared, sem):
    np_, d = shared.shape
    seg = np_ // NS
    c = lax.axis_index("c")
    s = lax.axis_index("s")
    w = c * NS + s
    nch, ch = idxs.shape
    zero16 = jnp.zeros((LANES,), jnp.float32)

    @pl.loop(0, ch)
    def _(j):
        for k in range(d // LANES):
            rows[j, pl.ds(k * LANES, LANES)] = zero16

    for k in range(seg // ch):
        pltpu.sync_copy(rows, shared.at[pl.ds(s * seg + k * ch, ch)])
    plsc.subcore_barrier()

    pltpu.sync_copy(src_hbm.at[w], idxs)
    pltpu.sync_copy(dst_hbm.at[w], idxd)

    @pl.loop(0, nch)
    def _(j):
        pltpu.async_copy(h_hbm.at[idxs.at[j]], rows, sem).wait()
        pltpu.sync_copy(rows, shared.at[idxd.at[j]], add=True)

    plsc.subcore_barrier()
    pltpu.sync_copy(shared.at[pl.ds(s * seg, seg)],
                    aggp_hbm.at[c, pl.ds(s * seg, seg)])


def _tc_norm_body(n, np_, x_ref, degp_ref, h_ref, nd_ref):
    do = degp_ref[0, 0, :] + degp_ref[0, 1, :]
    di = degp_ref[1, 0, :] + degp_ref[1, 1, :]
    ns_ = lax.rsqrt(jnp.maximum(do, 1.0))
    nd_ref[...] = lax.rsqrt(jnp.maximum(di, 1.0)).reshape(1, np_)
    h_ref[pl.ds(0, n), :] = x_ref[...] * ns_[:n][:, None]


def _tc_main_body(n, s_seq, l_seq, d,
                  aggp_ref, nd_ref, seq_ref, wc_ref, bc_ref, w1_ref, b1_ref,
                  wih_ref, whh_ref, bi_ref, bh_ref, w2_ref, b2_ref,
                  final_ref, outv, xv, xgv, ysv, btv):
    agg = aggp_ref[0, pl.ds(0, n), :] + aggp_ref[1, pl.ds(0, n), :]
    agg = agg * nd_ref[0, :n][:, None]
    med = jnp.dot(agg, wc_ref[...], preferred_element_type=jnp.float32) + bc_ref[...]
    outv[...] = jnp.dot(med, w1_ref[...], preferred_element_type=jnp.float32) + b1_ref[...]
    btv[...] = whh_ref[...].T
    bt = btv[...]
    bsum = bi_ref[...] + bh_ref[...]
    dn_t = (((1,), (1,)), ((), ()))

    def seq_body(si, _):
        def gather_body(t, _):
            idx = seq_ref[si, t]
            xv[pl.ds(t, 1), :] = outv[pl.ds(idx, 1), :]
            return 0
        lax.fori_loop(0, l_seq, gather_body, 0)
        xgv[...] = lax.dot_general(
            xv[...], wih_ref[...], dn_t,
            preferred_element_type=jnp.float32) + bsum

        def step_block(tb, hc):
            h, cc = hc
            for u in range(8):
                t = tb * 8 + u
                hc = h.reshape(d, 1)
                g = xgv[pl.ds(t, 1), :] + jnp.sum(
                    hc * bt, axis=0, keepdims=True)
                ig = jax.nn.sigmoid(g[:, 0:d])
                fg = jax.nn.sigmoid(g[:, d:2 * d])
                gg = jnp.tanh(g[:, 2 * d:3 * d])
                og = jax.nn.sigmoid(g[:, 3 * d:4 * d])
                cc = fg * cc + ig * gg
                h = og * jnp.tanh(cc)
                ysv[pl.ds(t, 1), :] = h
            return (h, cc)

        zero_h = jnp.zeros((1, d), jnp.float32)
        lax.fori_loop(0, l_seq // 8, step_block, (zero_h, zero_h))

        def scat_body(t, _):
            idx = seq_ref[si, t]
            outv[pl.ds(idx, 1), :] = ysv[pl.ds(t, 1), :]
            return 0
        lax.fori_loop(0, l_seq, scat_body, 0)
        return 0

    lax.fori_loop(0, s_seq, seq_body, 0)
    final_ref[...] = jnp.dot(outv[...], w2_ref[...],
                             preferred_element_type=jnp.float32) + b2_ref[...]


def kernel(in_feats, edge_index, seq_ids, W_conv, b_conv, W_ff1, b_ff1,
           W_ih, W_hh, b_ih, b_hh, W_ff2, b_ff2):
    n, d = in_feats.shape
    e = edge_index.shape[1]
    s_seq, l_seq = seq_ids.shape
    np_ = ((n + NS * LANES - 1) // (NS * LANES)) * (NS * LANES)
    epw = e // NW
    nch = epw // CH

    src3 = edge_index[0].reshape(NW, nch, CH)
    dst3 = edge_index[1].reshape(NW, nch, CH)

    mesh = plsc.VectorSubcoreMesh(core_axis_name="c", subcore_axis_name="s")
    seg = np_ // NS

    degp = pl.kernel(
        _sc_degrees,
        out_type=jax.ShapeDtypeStruct((2, 2, np_), jnp.float32),
        mesh=mesh,
        compiler_params=pltpu.CompilerParams(needs_layout_passes=False),
        scratch_types=[
            pltpu.VMEM((nch, CH), jnp.int32),
            pltpu.VMEM((np_,), jnp.float32),
            pltpu.VMEM((NS, seg), jnp.float32),
            pltpu.VMEM((seg,), jnp.float32),
            pltpu.VMEM_SHARED((NS, np_), jnp.float32),
        ],
    )(src3, dst3)

    h, norm_dst = pl.pallas_call(
        functools.partial(_tc_norm_body, n, np_),
        out_shape=(
            jax.ShapeDtypeStruct((np_, d), jnp.float32),
            jax.ShapeDtypeStruct((1, np_), jnp.float32),
        ),
    )(in_feats, degp)

    aggp = pl.kernel(
        _sc_agg,
        out_type=jax.ShapeDtypeStruct((2, np_, d), jnp.float32),
        mesh=mesh,
        scratch_types=[
            pltpu.VMEM((nch, CH), jnp.int32),
            pltpu.VMEM((nch, CH), jnp.int32),
            pltpu.VMEM((CH, d), jnp.float32),
            pltpu.VMEM_SHARED((np_, d), jnp.float32),
            pltpu.SemaphoreType.DMA,
        ],
    )(h, src3, dst3)

    final = pl.pallas_call(
        functools.partial(_tc_main_body, n, s_seq, l_seq, d),
        out_shape=jax.ShapeDtypeStruct((n, d), jnp.float32),
        in_specs=[
            pl.BlockSpec(memory_space=pltpu.VMEM),
            pl.BlockSpec(memory_space=pltpu.VMEM),
            pl.BlockSpec(memory_space=pltpu.SMEM),
        ] + [pl.BlockSpec(memory_space=pltpu.VMEM)] * 10,
        out_specs=pl.BlockSpec(memory_space=pltpu.VMEM),
        scratch_shapes=[
            pltpu.VMEM((n, d), jnp.float32),
            pltpu.VMEM((l_seq, d), jnp.float32),
            pltpu.VMEM((l_seq, 4 * d), jnp.float32),
            pltpu.VMEM((l_seq, d), jnp.float32),
            pltpu.VMEM((d, 4 * d), jnp.float32),
        ],
    )(aggp, norm_dst, seq_ids,
      W_conv, b_conv.reshape(1, d), W_ff1, b_ff1.reshape(1, d),
      W_ih, W_hh, b_ih.reshape(1, 4 * d), b_hh.reshape(1, 4 * d),
      W_ff2, b_ff2.reshape(1, d))
    return final


# speedup vs baseline: 1.4189x; 1.1121x over previous
import functools

import jax
import jax.numpy as jnp
from jax import lax
from jax.experimental import pallas as pl
from jax.experimental.pallas import tpu as pltpu
from jax.experimental.pallas import tpu_sc as plsc

NW = 32
NS = 16
LANES = 16
CH = 80


def _sc_degrees(src_hbm, dst_hbm, degp_hbm, idxv, hist):
    np_, = hist.shape
    c = lax.axis_index("c")
    s = lax.axis_index("s")
    w = c * NS + s
    epw, = idxv.shape
    zero16 = jnp.zeros((LANES,), jnp.float32)
    one16 = jnp.full((LANES,), 1.0, jnp.float32)
    for direction, eb in enumerate((src_hbm, dst_hbm)):
        @pl.loop(0, np_ // LANES)
        def _(i):
            hist[pl.ds(i * LANES, LANES)] = zero16

        pltpu.sync_copy(eb.at[w], idxv)

        @pl.loop(0, epw // LANES)
        def _(j):
            ids = idxv[pl.ds(j * LANES, LANES)]
            plsc.addupdate_scatter(hist, [ids], one16)

        pltpu.sync_copy(hist, degp_hbm.at[direction, w])


def _sc_agg(h_hbm, src_hbm, dst_hbm, aggp_hbm, idxs, idxd, rows2, shared, sem):
    np_, d = shared.shape
    seg = np_ // NS
    c = lax.axis_index("c")
    s = lax.axis_index("s")
    w = c * NS + s
    ch = idxs.shape[1]
    zero16 = jnp.zeros((LANES,), jnp.float32)

    @pl.loop(0, ch)
    def _(j):
        for k in range(d // LANES):
            rows2[0, j, pl.ds(k * LANES, LANES)] = zero16

    for k in range(seg // ch):
        pltpu.sync_copy(rows2.at[0], shared.at[pl.ds(s * seg + k * ch, ch)])
    plsc.subcore_barrier()

    nch = src_hbm.shape[1]
    bch = idxs.shape[0]

    @pl.loop(0, nch // bch)
    def _(b):
        pltpu.sync_copy(src_hbm.at[w, pl.ds(b * bch, bch)], idxs)
        pltpu.sync_copy(dst_hbm.at[w, pl.ds(b * bch, bch)], idxd)
        pltpu.async_copy(h_hbm.at[idxs.at[0]], rows2.at[0], sem.at[0])

        @pl.loop(0, bch)
        def _(j):
            p = lax.rem(j, 2)
            q = 1 - p

            @pl.when(j + 1 < bch)
            def _():
                pltpu.async_copy(h_hbm.at[idxs.at[j + 1]], rows2.at[q],
                                 sem.at[q])

            pltpu.make_async_copy(h_hbm.at[idxs.at[j]], rows2.at[p],
                                  sem.at[p]).wait()
            pltpu.sync_copy(rows2.at[p], shared.at[idxd.at[j]], add=True)

    plsc.subcore_barrier()
    pltpu.sync_copy(shared.at[pl.ds(s * seg, seg)],
                    aggp_hbm.at[c, pl.ds(s * seg, seg)])


def _tc_norm_body(n, np_, x_ref, degp_ref, h_ref, nd_ref):
    do = jnp.sum(degp_ref[0], axis=0)
    di = jnp.sum(degp_ref[1], axis=0)
    ns_ = lax.rsqrt(jnp.maximum(do, 1.0))
    nd_ref[...] = lax.rsqrt(jnp.maximum(di, 1.0)).reshape(1, np_)
    h_ref[pl.ds(0, n), :] = x_ref[...] * ns_[:n][:, None]


def _tc_main_body(n, s_seq, l_seq, d,
                  aggp_ref, nd_ref, seq_ref, wc_ref, bc_ref, w1_ref, b1_ref,
                  wih_ref, whh_ref, bi_ref, bh_ref, w2_ref, b2_ref,
                  final_ref, outv, xv, xgv, ysv, btv):
    agg = aggp_ref[0, pl.ds(0, n), :] + aggp_ref[1, pl.ds(0, n), :]
    agg = agg * nd_ref[0, :n][:, None]
    med = jnp.dot(agg, wc_ref[...], preferred_element_type=jnp.float32) + bc_ref[...]
    outv[...] = jnp.dot(med, w1_ref[...], preferred_element_type=jnp.float32) + b1_ref[...]
    btv[...] = whh_ref[...].T
    bt = btv[...]
    bsum = bi_ref[...] + bh_ref[...]
    dn_t = (((1,), (1,)), ((), ()))

    def seq_body(si, _):
        def gather_body(t, _):
            idx = seq_ref[si, t]
            xv[pl.ds(t, 1), :] = outv[pl.ds(idx, 1), :]
            return 0
        lax.fori_loop(0, l_seq, gather_body, 0)
        xgv[...] = lax.dot_general(
            xv[...], wih_ref[...], dn_t,
            preferred_element_type=jnp.float32) + bsum

        def step_block(tb, hc):
            h, cc = hc
            for u in range(8):
                t = tb * 8 + u
                hc = h.reshape(d, 1)
                g = xgv[pl.ds(t, 1), :] + jnp.sum(
                    hc * bt, axis=0, keepdims=True)
                ig = jax.nn.sigmoid(g[:, 0:d])
                fg = jax.nn.sigmoid(g[:, d:2 * d])
                gg = jnp.tanh(g[:, 2 * d:3 * d])
                og = jax.nn.sigmoid(g[:, 3 * d:4 * d])
                cc = fg * cc + ig * gg
                h = og * jnp.tanh(cc)
                ysv[pl.ds(t, 1), :] = h
            return (h, cc)

        zero_h = jnp.zeros((1, d), jnp.float32)
        lax.fori_loop(0, l_seq // 8, step_block, (zero_h, zero_h))

        def scat_body(t, _):
            idx = seq_ref[si, t]
            outv[pl.ds(idx, 1), :] = ysv[pl.ds(t, 1), :]
            return 0
        lax.fori_loop(0, l_seq, scat_body, 0)
        return 0

    lax.fori_loop(0, s_seq, seq_body, 0)
    final_ref[...] = jnp.dot(outv[...], w2_ref[...],
                             preferred_element_type=jnp.float32) + b2_ref[...]


def kernel(in_feats, edge_index, seq_ids, W_conv, b_conv, W_ff1, b_ff1,
           W_ih, W_hh, b_ih, b_hh, W_ff2, b_ff2):
    n, d = in_feats.shape
    e = edge_index.shape[1]
    s_seq, l_seq = seq_ids.shape
    np_ = ((n + NS * LANES - 1) // (NS * LANES)) * (NS * LANES)
    epw = e // NW
    nch = epw // CH

    src2 = edge_index[0].reshape(NW, epw)
    dst2 = edge_index[1].reshape(NW, epw)
    epw_p = ((epw + 8 * CH - 1) // (8 * CH)) * (8 * CH)
    npad = epw_p - epw
    nchp = epw_p // CH
    padb = jnp.broadcast_to(
        n + (jnp.arange(npad, dtype=jnp.int32) % (np_ - n)), (NW, npad))
    src3 = jnp.concatenate([src2, padb], axis=1).reshape(NW, nchp, CH)
    dst3 = jnp.concatenate([dst2, padb], axis=1).reshape(NW, nchp, CH)

    mesh = plsc.VectorSubcoreMesh(core_axis_name="c", subcore_axis_name="s")
    seg = np_ // NS

    degp = pl.kernel(
        _sc_degrees,
        out_type=jax.ShapeDtypeStruct((2, NW, np_), jnp.float32),
        mesh=mesh,
        compiler_params=pltpu.CompilerParams(needs_layout_passes=False),
        scratch_types=[
            pltpu.VMEM((epw,), jnp.int32),
            pltpu.VMEM((np_,), jnp.float32),
        ],
    )(src2, dst2)

    h, norm_dst = pl.pallas_call(
        functools.partial(_tc_norm_body, n, np_),
        out_shape=(
            jax.ShapeDtypeStruct((np_, d), jnp.float32),
            jax.ShapeDtypeStruct((1, np_), jnp.float32),
        ),
    )(in_feats, degp)

    aggp = pl.kernel(
        _sc_agg,
        out_type=jax.ShapeDtypeStruct((2, np_, d), jnp.float32),
        mesh=mesh,
        scratch_types=[
            pltpu.VMEM((32, CH), jnp.int32),
            pltpu.VMEM((32, CH), jnp.int32),
            pltpu.VMEM((2, CH, d), jnp.float32),
            pltpu.VMEM_SHARED((np_, d), jnp.float32),
            pltpu.SemaphoreType.DMA((2,)),
        ],
    )(h, src3, dst3)

    final = pl.pallas_call(
        functools.partial(_tc_main_body, n, s_seq, l_seq, d),
        out_shape=jax.ShapeDtypeStruct((n, d), jnp.float32),
        in_specs=[
            pl.BlockSpec(memory_space=pltpu.VMEM),
            pl.BlockSpec(memory_space=pltpu.VMEM),
            pl.BlockSpec(memory_space=pltpu.SMEM),
        ] + [pl.BlockSpec(memory_space=pltpu.VMEM)] * 10,
        out_specs=pl.BlockSpec(memory_space=pltpu.VMEM),
        scratch_shapes=[
            pltpu.VMEM((n, d), jnp.float32),
            pltpu.VMEM((l_seq, d), jnp.float32),
            pltpu.VMEM((l_seq, 4 * d), jnp.float32),
            pltpu.VMEM((l_seq, d), jnp.float32),
            pltpu.VMEM((d, 4 * d), jnp.float32),
        ],
    )(aggp, norm_dst, seq_ids,
      W_conv, b_conv.reshape(1, d), W_ff1, b_ff1.reshape(1, d),
      W_ih, W_hh, b_ih.reshape(1, 4 * d), b_hh.reshape(1, 4 * d),
      W_ff2, b_ff2.reshape(1, d))
    return final


# speedup vs baseline: 1.4801x; 1.0432x over previous
import functools

import jax
import jax.numpy as jnp
from jax import lax
from jax.experimental import pallas as pl
from jax.experimental.pallas import tpu as pltpu
from jax.experimental.pallas import tpu_sc as plsc

NW = 32
NS = 16
LANES = 16
CH = 80


def _sc_degrees(src_hbm, dst_hbm, degp_hbm, idxv, hist):
    np_, = hist.shape
    c = lax.axis_index("c")
    s = lax.axis_index("s")
    w = c * NS + s
    epw, = idxv.shape
    zero16 = jnp.zeros((LANES,), jnp.float32)
    one16 = jnp.full((LANES,), 1.0, jnp.float32)
    for direction, eb in enumerate((src_hbm, dst_hbm)):
        @pl.loop(0, np_ // LANES)
        def _(i):
            hist[pl.ds(i * LANES, LANES)] = zero16

        pltpu.sync_copy(eb.at[w], idxv)

        @pl.loop(0, epw // LANES)
        def _(j):
            ids = idxv[pl.ds(j * LANES, LANES)]
            plsc.addupdate_scatter(hist, [ids], one16)

        pltpu.sync_copy(hist, degp_hbm.at[direction, w])


def _sc_agg(h_hbm, src_hbm, dst_hbm, aggp_hbm, idxs, idxd, rows2, shared, sem):
    np_, d = shared.shape
    seg = np_ // NS
    c = lax.axis_index("c")
    s = lax.axis_index("s")
    w = c * NS + s
    ch = idxs.shape[1]
    zero16 = jnp.zeros((LANES,), jnp.float32)

    @pl.loop(0, ch)
    def _(j):
        for k in range(d // LANES):
            rows2[0, j, pl.ds(k * LANES, LANES)] = zero16

    for k in range(seg // ch):
        pltpu.sync_copy(rows2.at[0], shared.at[pl.ds(s * seg + k * ch, ch)])
    plsc.subcore_barrier()

    nch = src_hbm.shape[1]
    bch = idxs.shape[0]

    @pl.loop(0, nch // bch)
    def _(b):
        pltpu.sync_copy(src_hbm.at[w, pl.ds(b * bch, bch)], idxs)
        pltpu.sync_copy(dst_hbm.at[w, pl.ds(b * bch, bch)], idxd)
        pltpu.async_copy(h_hbm.at[idxs.at[0]], rows2.at[0], sem.at[0])

        @pl.loop(0, bch)
        def _(j):
            p = lax.rem(j, 2)
            q = 1 - p

            @pl.when(j + 1 < bch)
            def _():
                pltpu.async_copy(h_hbm.at[idxs.at[j + 1]], rows2.at[q],
                                 sem.at[q])

            pltpu.make_async_copy(h_hbm.at[idxs.at[j]], rows2.at[p],
                                  sem.at[p]).wait()
            pltpu.sync_copy(rows2.at[p], shared.at[idxd.at[j]], add=True)

    plsc.subcore_barrier()
    pltpu.sync_copy(shared.at[pl.ds(s * seg, seg)],
                    aggp_hbm.at[c, pl.ds(s * seg, seg)])


def _tc_norm_body(n, np_, x_ref, degp_ref, h_ref, nd_ref):
    do = jnp.sum(degp_ref[0], axis=0)
    di = jnp.sum(degp_ref[1], axis=0)
    ns_ = lax.rsqrt(jnp.maximum(do, 1.0))
    nd_ref[...] = lax.rsqrt(jnp.maximum(di, 1.0)).reshape(1, np_)
    h_ref[pl.ds(0, n), :] = x_ref[...] * ns_[:n][:, None]


def _tc_main_body(n, s_seq, l_seq, d,
                  aggp_ref, nd_ref, seq_ref, wc_ref, bc_ref, w1_ref, b1_ref,
                  wih_ref, whh_ref, bi_ref, bh_ref, w2_ref, b2_ref,
                  final_ref, outv, xv, xgv, btv):
    agg = aggp_ref[0, pl.ds(0, n), :] + aggp_ref[1, pl.ds(0, n), :]
    agg = agg * nd_ref[0, :n][:, None]
    med = jnp.dot(agg, wc_ref[...], preferred_element_type=jnp.float32) + bc_ref[...]
    outv[...] = jnp.dot(med, w1_ref[...], preferred_element_type=jnp.float32) + b1_ref[...]
    btv[...] = whh_ref[...].T
    bt = btv[...]
    bsum = bi_ref[...] + bh_ref[...]
    dn_t = (((1,), (1,)), ((), ()))

    def seq_body(si, _):
        def gather_body(t, _):
            idx = seq_ref[si, t]
            xv[pl.ds(t, 1), :] = outv[pl.ds(idx, 1), :]
            return 0
        lax.fori_loop(0, l_seq, gather_body, 0)
        xgv[...] = lax.dot_general(
            xv[...], wih_ref[...], dn_t,
            preferred_element_type=jnp.float32) + bsum

        def step_block(tb, hc):
            h, cc = hc
            for u in range(8):
                t = tb * 8 + u
                hc = h.reshape(d, 1)
                g = xgv[pl.ds(t, 1), :] + jnp.sum(
                    hc * bt, axis=0, keepdims=True)
                ig = jax.nn.sigmoid(g[:, 0:d])
                fg = jax.nn.sigmoid(g[:, d:2 * d])
                gg = jnp.tanh(g[:, 2 * d:3 * d])
                og = jax.nn.sigmoid(g[:, 3 * d:4 * d])
                cc = fg * cc + ig * gg
                h = og * jnp.tanh(cc)
                oidx = seq_ref[si, t]
                outv[pl.ds(oidx, 1), :] = h
            return (h, cc)

        zero_h = jnp.zeros((1, d), jnp.float32)
        lax.fori_loop(0, l_seq // 8, step_block, (zero_h, zero_h))
        return 0

    lax.fori_loop(0, s_seq, seq_body, 0)
    final_ref[...] = jnp.dot(outv[...], w2_ref[...],
                             preferred_element_type=jnp.float32) + b2_ref[...]


def kernel(in_feats, edge_index, seq_ids, W_conv, b_conv, W_ff1, b_ff1,
           W_ih, W_hh, b_ih, b_hh, W_ff2, b_ff2):
    n, d = in_feats.shape
    e = edge_index.shape[1]
    s_seq, l_seq = seq_ids.shape
    np_ = ((n + NS * LANES - 1) // (NS * LANES)) * (NS * LANES)
    epw = e // NW
    nch = epw // CH

    src2 = edge_index[0].reshape(NW, epw)
    dst2 = edge_index[1].reshape(NW, epw)
    epw_p = ((epw + 8 * CH - 1) // (8 * CH)) * (8 * CH)
    npad = epw_p - epw
    nchp = epw_p // CH
    padb = jnp.broadcast_to(
        n + (jnp.arange(npad, dtype=jnp.int32) % (np_ - n)), (NW, npad))
    src3 = jnp.concatenate([src2, padb], axis=1).reshape(NW, nchp, CH)
    dst3 = jnp.concatenate([dst2, padb], axis=1).reshape(NW, nchp, CH)

    mesh = plsc.VectorSubcoreMesh(core_axis_name="c", subcore_axis_name="s")
    seg = np_ // NS

    degp = pl.kernel(
        _sc_degrees,
        out_type=jax.ShapeDtypeStruct((2, NW, np_), jnp.float32),
        mesh=mesh,
        compiler_params=pltpu.CompilerParams(needs_layout_passes=False),
        scratch_types=[
            pltpu.VMEM((epw,), jnp.int32),
            pltpu.VMEM((np_,), jnp.float32),
        ],
    )(src2, dst2)

    h, norm_dst = pl.pallas_call(
        functools.partial(_tc_norm_body, n, np_),
        out_shape=(
            jax.ShapeDtypeStruct((np_, d), jnp.float32),
            jax.ShapeDtypeStruct((1, np_), jnp.float32),
        ),
    )(in_feats, degp)

    aggp = pl.kernel(
        _sc_agg,
        out_type=jax.ShapeDtypeStruct((2, np_, d), jnp.float32),
        mesh=mesh,
        scratch_types=[
            pltpu.VMEM((32, CH), jnp.int32),
            pltpu.VMEM((32, CH), jnp.int32),
            pltpu.VMEM((2, CH, d), jnp.float32),
            pltpu.VMEM_SHARED((np_, d), jnp.float32),
            pltpu.SemaphoreType.DMA((2,)),
        ],
    )(h, src3, dst3)

    final = pl.pallas_call(
        functools.partial(_tc_main_body, n, s_seq, l_seq, d),
        out_shape=jax.ShapeDtypeStruct((n, d), jnp.float32),
        in_specs=[
            pl.BlockSpec(memory_space=pltpu.VMEM),
            pl.BlockSpec(memory_space=pltpu.VMEM),
            pl.BlockSpec(memory_space=pltpu.SMEM),
        ] + [pl.BlockSpec(memory_space=pltpu.VMEM)] * 10,
        out_specs=pl.BlockSpec(memory_space=pltpu.VMEM),
        scratch_shapes=[
            pltpu.VMEM((n, d), jnp.float32),
            pltpu.VMEM((l_seq, d), jnp.float32),
            pltpu.VMEM((l_seq, 4 * d), jnp.float32),
            pltpu.VMEM((d, 4 * d), jnp.float32),
        ],
    )(aggp, norm_dst, seq_ids,
      W_conv, b_conv.reshape(1, d), W_ff1, b_ff1.reshape(1, d),
      W_ih, W_hh, b_ih.reshape(1, 4 * d), b_hh.reshape(1, 4 * d),
      W_ff2, b_ff2.reshape(1, d))
    return final


# speedup vs baseline: 1.4973x; 1.0116x over previous
import functools

import jax
import jax.numpy as jnp
from jax import lax
from jax.experimental import pallas as pl
from jax.experimental.pallas import tpu as pltpu
from jax.experimental.pallas import tpu_sc as plsc

NW = 32
NS = 16
LANES = 16
CH = 128


def _sc_degrees(src_hbm, dst_hbm, degp_hbm, idxv, hist):
    np_, = hist.shape
    c = lax.axis_index("c")
    s = lax.axis_index("s")
    w = c * NS + s
    epw, = idxv.shape
    zero16 = jnp.zeros((LANES,), jnp.float32)
    one16 = jnp.full((LANES,), 1.0, jnp.float32)
    for direction, eb in enumerate((src_hbm, dst_hbm)):
        @pl.loop(0, np_ // LANES)
        def _(i):
            hist[pl.ds(i * LANES, LANES)] = zero16

        pltpu.sync_copy(eb.at[w], idxv)

        @pl.loop(0, epw // LANES)
        def _(j):
            ids = idxv[pl.ds(j * LANES, LANES)]
            plsc.addupdate_scatter(hist, [ids], one16)

        pltpu.sync_copy(hist, degp_hbm.at[direction, w])


def _sc_agg(h_hbm, src_hbm, dst_hbm, aggp_hbm, idxs, idxd, rows2, shared, sem):
    np_, d = shared.shape
    seg = np_ // NS
    c = lax.axis_index("c")
    s = lax.axis_index("s")
    w = c * NS + s
    ch = idxs.shape[1]
    zero16 = jnp.zeros((LANES,), jnp.float32)

    @pl.loop(0, ch)
    def _(j):
        for k in range(d // LANES):
            rows2[0, j, pl.ds(k * LANES, LANES)] = zero16

    for k in range(seg // ch):
        pltpu.sync_copy(rows2.at[0], shared.at[pl.ds(s * seg + k * ch, ch)])
    plsc.subcore_barrier()

    nch = src_hbm.shape[1]
    bch = idxs.shape[0]

    @pl.loop(0, nch // bch)
    def _(b):
        pltpu.sync_copy(src_hbm.at[w, pl.ds(b * bch, bch)], idxs)
        pltpu.sync_copy(dst_hbm.at[w, pl.ds(b * bch, bch)], idxd)
        pltpu.async_copy(h_hbm.at[idxs.at[0]], rows2.at[0], sem.at[0])

        @pl.loop(0, bch)
        def _(j):
            p = lax.rem(j, 2)
            q = 1 - p

            @pl.when(j + 1 < bch)
            def _():
                pltpu.async_copy(h_hbm.at[idxs.at[j + 1]], rows2.at[q],
                                 sem.at[q])

            pltpu.make_async_copy(h_hbm.at[idxs.at[j]], rows2.at[p],
                                  sem.at[p]).wait()
            pltpu.sync_copy(rows2.at[p], shared.at[idxd.at[j]], add=True)

    plsc.subcore_barrier()
    pltpu.sync_copy(shared.at[pl.ds(s * seg, seg)],
                    aggp_hbm.at[c, pl.ds(s * seg, seg)])


def _tc_norm_body(n, np_, x_ref, degp_ref, h_ref, nd_ref):
    do = jnp.sum(degp_ref[0], axis=0)
    di = jnp.sum(degp_ref[1], axis=0)
    ns_ = lax.rsqrt(jnp.maximum(do, 1.0))
    nd_ref[...] = lax.rsqrt(jnp.maximum(di, 1.0)).reshape(1, np_)
    h_ref[pl.ds(0, n), :] = x_ref[...] * ns_[:n][:, None]


def _tc_main_body(n, s_seq, l_seq, d,
                  aggp_ref, nd_ref, seq_ref, wc_ref, bc_ref, w1_ref, b1_ref,
                  wih_ref, whh_ref, bi_ref, bh_ref, w2_ref, b2_ref,
                  final_ref, outv, xv, xgv, btv):
    agg = aggp_ref[0, pl.ds(0, n), :] + aggp_ref[1, pl.ds(0, n), :]
    agg = agg * nd_ref[0, :n][:, None]
    med = jnp.dot(agg, wc_ref[...], preferred_element_type=jnp.float32) + bc_ref[...]
    outv[...] = jnp.dot(med, w1_ref[...], preferred_element_type=jnp.float32) + b1_ref[...]
    btv[...] = whh_ref[...].T
    bt = btv[...]
    bsum = bi_ref[...] + bh_ref[...]
    dn_t = (((1,), (1,)), ((), ()))

    def seq_body(si, _):
        def gather_body(t, _):
            idx = seq_ref[si, t]
            xv[pl.ds(t, 1), :] = outv[pl.ds(idx, 1), :]
            return 0
        lax.fori_loop(0, l_seq, gather_body, 0)
        xgv[...] = lax.dot_general(
            xv[...], wih_ref[...], dn_t,
            preferred_element_type=jnp.float32) + bsum

        def step_block(tb, hc):
            h, cc = hc
            for u in range(8):
                t = tb * 8 + u
                hc = h.reshape(d, 1)
                g = xgv[pl.ds(t, 1), :] + jnp.sum(
                    hc * bt, axis=0, keepdims=True)
                ig = jax.nn.sigmoid(g[:, 0:d])
                fg = jax.nn.sigmoid(g[:, d:2 * d])
                gg = jnp.tanh(g[:, 2 * d:3 * d])
                og = jax.nn.sigmoid(g[:, 3 * d:4 * d])
                cc = fg * cc + ig * gg
                h = og * jnp.tanh(cc)
                oidx = seq_ref[si, t]
                outv[pl.ds(oidx, 1), :] = h
            return (h, cc)

        zero_h = jnp.zeros((1, d), jnp.float32)
        lax.fori_loop(0, l_seq // 8, step_block, (zero_h, zero_h))
        return 0

    lax.fori_loop(0, s_seq, seq_body, 0)
    final_ref[...] = jnp.dot(outv[...], w2_ref[...],
                             preferred_element_type=jnp.float32) + b2_ref[...]


def kernel(in_feats, edge_index, seq_ids, W_conv, b_conv, W_ff1, b_ff1,
           W_ih, W_hh, b_ih, b_hh, W_ff2, b_ff2):
    n, d = in_feats.shape
    e = edge_index.shape[1]
    s_seq, l_seq = seq_ids.shape
    np_ = ((n + NS * LANES - 1) // (NS * LANES)) * (NS * LANES)
    epw = e // NW
    nch = epw // CH

    src2 = edge_index[0].reshape(NW, epw)
    dst2 = edge_index[1].reshape(NW, epw)
    epw_p = ((epw + 8 * CH - 1) // (8 * CH)) * (8 * CH)
    npad = epw_p - epw
    nchp = epw_p // CH
    padb = jnp.broadcast_to(
        n + (jnp.arange(npad, dtype=jnp.int32) % (np_ - n)), (NW, npad))
    src3 = jnp.concatenate([src2, padb], axis=1).reshape(NW, nchp, CH)
    dst3 = jnp.concatenate([dst2, padb], axis=1).reshape(NW, nchp, CH)

    mesh = plsc.VectorSubcoreMesh(core_axis_name="c", subcore_axis_name="s")
    seg = np_ // NS

    degp = pl.kernel(
        _sc_degrees,
        out_type=jax.ShapeDtypeStruct((2, NW, np_), jnp.float32),
        mesh=mesh,
        compiler_params=pltpu.CompilerParams(needs_layout_passes=False),
        scratch_types=[
            pltpu.VMEM((epw,), jnp.int32),
            pltpu.VMEM((np_,), jnp.float32),
        ],
    )(src2, dst2)

    h, norm_dst = pl.pallas_call(
        functools.partial(_tc_norm_body, n, np_),
        out_shape=(
            jax.ShapeDtypeStruct((np_, d), jnp.float32),
            jax.ShapeDtypeStruct((1, np_), jnp.float32),
        ),
    )(in_feats, degp)

    aggp = pl.kernel(
        _sc_agg,
        out_type=jax.ShapeDtypeStruct((2, np_, d), jnp.float32),
        mesh=mesh,
        scratch_types=[
            pltpu.VMEM((16, CH), jnp.int32),
            pltpu.VMEM((16, CH), jnp.int32),
            pltpu.VMEM((2, CH, d), jnp.float32),
            pltpu.VMEM_SHARED((np_, d), jnp.float32),
            pltpu.SemaphoreType.DMA((2,)),
        ],
    )(h, src3, dst3)

    final = pl.pallas_call(
        functools.partial(_tc_main_body, n, s_seq, l_seq, d),
        out_shape=jax.ShapeDtypeStruct((n, d), jnp.float32),
        in_specs=[
            pl.BlockSpec(memory_space=pltpu.VMEM),
            pl.BlockSpec(memory_space=pltpu.VMEM),
            pl.BlockSpec(memory_space=pltpu.SMEM),
        ] + [pl.BlockSpec(memory_space=pltpu.VMEM)] * 10,
        out_specs=pl.BlockSpec(memory_space=pltpu.VMEM),
        scratch_shapes=[
            pltpu.VMEM((n, d), jnp.float32),
            pltpu.VMEM((l_seq, d), jnp.float32),
            pltpu.VMEM((l_seq, 4 * d), jnp.float32),
            pltpu.VMEM((d, 4 * d), jnp.float32),
        ],
    )(aggp, norm_dst, seq_ids,
      W_conv, b_conv.reshape(1, d), W_ff1, b_ff1.reshape(1, d),
      W_ih, W_hh, b_ih.reshape(1, 4 * d), b_hh.reshape(1, 4 * d),
      W_ff2, b_ff2.reshape(1, d))
    return final


# speedup vs baseline: 1.5335x; 1.0242x over previous
import functools

import jax
import jax.numpy as jnp
from jax import lax
from jax.experimental import pallas as pl
from jax.experimental.pallas import tpu as pltpu
from jax.experimental.pallas import tpu_sc as plsc

NW = 32
NS = 16
LANES = 16
CH = 128


def _sc_degrees(src_hbm, dst_hbm, degp_hbm, idxv, hist):
    np_, = hist.shape
    c = lax.axis_index("c")
    s = lax.axis_index("s")
    w = c * NS + s
    epw, = idxv.shape
    zero16 = jnp.zeros((LANES,), jnp.float32)
    one16 = jnp.full((LANES,), 1.0, jnp.float32)
    for direction, eb in enumerate((src_hbm, dst_hbm)):
        @pl.loop(0, np_ // LANES)
        def _(i):
            hist[pl.ds(i * LANES, LANES)] = zero16

        pltpu.sync_copy(eb.at[w], idxv)

        @pl.loop(0, epw // LANES)
        def _(j):
            ids = idxv[pl.ds(j * LANES, LANES)]
            plsc.addupdate_scatter(hist, [ids], one16)

        pltpu.sync_copy(hist, degp_hbm.at[direction, w])


def _sc_agg(h_hbm, src_hbm, dst_hbm, aggp_hbm, idxs, idxd, rows2, shared, sem):
    np_, d = shared.shape
    seg = np_ // NS
    c = lax.axis_index("c")
    s = lax.axis_index("s")
    w = c * NS + s
    ch = idxs.shape[1]
    zero16 = jnp.zeros((LANES,), jnp.float32)

    @pl.loop(0, ch)
    def _(j):
        for k in range(d // LANES):
            rows2[0, j, pl.ds(k * LANES, LANES)] = zero16

    for k in range(seg // ch):
        pltpu.sync_copy(rows2.at[0], shared.at[pl.ds(s * seg + k * ch, ch)])
    plsc.subcore_barrier()

    nch = src_hbm.shape[1]
    bch = idxs.shape[0]

    @pl.loop(0, nch // bch)
    def _(b):
        pltpu.sync_copy(src_hbm.at[w, pl.ds(b * bch, bch)], idxs)
        pltpu.sync_copy(dst_hbm.at[w, pl.ds(b * bch, bch)], idxd)
        pltpu.async_copy(h_hbm.at[idxs.at[0]], rows2.at[0], sem.at[0])

        @pl.loop(0, bch)
        def _(j):
            p = lax.rem(j, 2)
            q = 1 - p

            @pl.when(j + 1 < bch)
            def _():
                pltpu.async_copy(h_hbm.at[idxs.at[j + 1]], rows2.at[q],
                                 sem.at[q])

            pltpu.make_async_copy(h_hbm.at[idxs.at[j]], rows2.at[p],
                                  sem.at[p]).wait()
            pltpu.sync_copy(rows2.at[p], shared.at[idxd.at[j]], add=True)

    plsc.subcore_barrier()
    pltpu.sync_copy(shared.at[pl.ds(s * seg, seg)],
                    aggp_hbm.at[c, pl.ds(s * seg, seg)])


def _tc_norm_body(n, np_, x_ref, degp_ref, h_ref, nd_ref):
    do = jnp.sum(degp_ref[0], axis=0)
    di = jnp.sum(degp_ref[1], axis=0)
    ns_ = lax.rsqrt(jnp.maximum(do, 1.0))
    nd_ref[...] = lax.rsqrt(jnp.maximum(di, 1.0)).reshape(1, np_)
    h_ref[pl.ds(0, n), :] = x_ref[...] * ns_[:n][:, None]


def _tc_main_body(n, s_seq, l_seq, d,
                  aggp_ref, nd_ref, seq_ref, wc_ref, bc_ref, w1_ref, b1_ref,
                  wih_ref, whh_ref, bi_ref, bh_ref, w2_ref, b2_ref,
                  final_ref, outv, xv, xgv, btv):
    agg = aggp_ref[0, pl.ds(0, n), :] + aggp_ref[1, pl.ds(0, n), :]
    agg = agg * nd_ref[0, :n][:, None]
    med = jnp.dot(agg, wc_ref[...], preferred_element_type=jnp.float32) + bc_ref[...]
    outv[...] = jnp.dot(med, w1_ref[...], preferred_element_type=jnp.float32) + b1_ref[...]
    btv[...] = whh_ref[...].T
    bt = btv[...]
    bsum = bi_ref[...] + bh_ref[...]
    dn_t = (((1,), (1,)), ((), ()))

    def seq_body(si, _):
        def gather_body(tb, _):
            for u in range(4):
                t = tb * 4 + u
                idx = seq_ref[si, t]
                xv[pl.ds(t, 1), :] = outv[pl.ds(idx, 1), :]
            return 0
        lax.fori_loop(0, l_seq // 4, gather_body, 0)
        xgv[...] = lax.dot_general(
            xv[...], wih_ref[...], dn_t,
            preferred_element_type=jnp.float32) + bsum

        def step_block(tb, hc):
            h, cc = hc
            for u in range(8):
                t = tb * 8 + u
                hc = h.reshape(d, 1)
                g = xgv[pl.ds(t, 1), :] + jnp.sum(
                    hc * bt, axis=0, keepdims=True)
                ig = jax.nn.sigmoid(g[:, 0:d])
                fg = jax.nn.sigmoid(g[:, d:2 * d])
                gg = jnp.tanh(g[:, 2 * d:3 * d])
                og = jax.nn.sigmoid(g[:, 3 * d:4 * d])
                cc = fg * cc + ig * gg
                h = og * jnp.tanh(cc)
                oidx = seq_ref[si, t]
                outv[pl.ds(oidx, 1), :] = h
            return (h, cc)

        zero_h = jnp.zeros((1, d), jnp.float32)
        lax.fori_loop(0, l_seq // 8, step_block, (zero_h, zero_h))
        return 0

    lax.fori_loop(0, s_seq, seq_body, 0)
    final_ref[...] = jnp.dot(outv[...], w2_ref[...],
                             preferred_element_type=jnp.float32) + b2_ref[...]


def kernel(in_feats, edge_index, seq_ids, W_conv, b_conv, W_ff1, b_ff1,
           W_ih, W_hh, b_ih, b_hh, W_ff2, b_ff2):
    n, d = in_feats.shape
    e = edge_index.shape[1]
    s_seq, l_seq = seq_ids.shape
    np_ = ((n + NS * LANES - 1) // (NS * LANES)) * (NS * LANES)
    epw = e // NW
    nch = epw // CH

    src2 = edge_index[0].reshape(NW, epw)
    dst2 = edge_index[1].reshape(NW, epw)
    epw_p = ((epw + 8 * CH - 1) // (8 * CH)) * (8 * CH)
    npad = epw_p - epw
    nchp = epw_p // CH
    padb = jnp.broadcast_to(
        n + (jnp.arange(npad, dtype=jnp.int32) % (np_ - n)), (NW, npad))
    src3 = jnp.concatenate([src2, padb], axis=1).reshape(NW, nchp, CH)
    dst3 = jnp.concatenate([dst2, padb], axis=1).reshape(NW, nchp, CH)

    mesh = plsc.VectorSubcoreMesh(core_axis_name="c", subcore_axis_name="s")
    seg = np_ // NS

    degp = pl.kernel(
        _sc_degrees,
        out_type=jax.ShapeDtypeStruct((2, NW, np_), jnp.float32),
        mesh=mesh,
        compiler_params=pltpu.CompilerParams(needs_layout_passes=False),
        scratch_types=[
            pltpu.VMEM((epw,), jnp.int32),
            pltpu.VMEM((np_,), jnp.float32),
        ],
    )(src2, dst2)

    h, norm_dst = pl.pallas_call(
        functools.partial(_tc_norm_body, n, np_),
        out_shape=(
            jax.ShapeDtypeStruct((np_, d), jnp.float32),
            jax.ShapeDtypeStruct((1, np_), jnp.float32),
        ),
    )(in_feats, degp)

    aggp = pl.kernel(
        _sc_agg,
        out_type=jax.ShapeDtypeStruct((2, np_, d), jnp.float32),
        mesh=mesh,
        scratch_types=[
            pltpu.VMEM((16, CH), jnp.int32),
            pltpu.VMEM((16, CH), jnp.int32),
            pltpu.VMEM((2, CH, d), jnp.float32),
            pltpu.VMEM_SHARED((np_, d), jnp.float32),
            pltpu.SemaphoreType.DMA((2,)),
        ],
    )(h, src3, dst3)

    final = pl.pallas_call(
        functools.partial(_tc_main_body, n, s_seq, l_seq, d),
        out_shape=jax.ShapeDtypeStruct((n, d), jnp.float32),
        in_specs=[
            pl.BlockSpec(memory_space=pltpu.VMEM),
            pl.BlockSpec(memory_space=pltpu.VMEM),
            pl.BlockSpec(memory_space=pltpu.SMEM),
        ] + [pl.BlockSpec(memory_space=pltpu.VMEM)] * 10,
        out_specs=pl.BlockSpec(memory_space=pltpu.VMEM),
        scratch_shapes=[
            pltpu.VMEM((n, d), jnp.float32),
            pltpu.VMEM((l_seq, d), jnp.float32),
            pltpu.VMEM((l_seq, 4 * d), jnp.float32),
            pltpu.VMEM((d, 4 * d), jnp.float32),
        ],
    )(aggp, norm_dst, seq_ids,
      W_conv, b_conv.reshape(1, d), W_ff1, b_ff1.reshape(1, d),
      W_ih, W_hh, b_ih.reshape(1, 4 * d), b_hh.reshape(1, 4 * d),
      W_ff2, b_ff2.reshape(1, d))
    return final


# speedup vs baseline: 1.5794x; 1.0299x over previous
import functools

import jax
import jax.numpy as jnp
from jax import lax
from jax.experimental import pallas as pl
from jax.experimental.pallas import tpu as pltpu
from jax.experimental.pallas import tpu_sc as plsc

NW = 32
NS = 16
LANES = 16
CH = 128


def _sc_degrees(src_hbm, dst_hbm, degp_hbm, idxv, hist):
    np_, = hist.shape
    c = lax.axis_index("c")
    s = lax.axis_index("s")
    w = c * NS + s
    epw, = idxv.shape
    zero16 = jnp.zeros((LANES,), jnp.float32)
    one16 = jnp.full((LANES,), 1.0, jnp.float32)
    for direction, eb in enumerate((src_hbm, dst_hbm)):
        @pl.loop(0, np_ // LANES)
        def _(i):
            hist[pl.ds(i * LANES, LANES)] = zero16

        pltpu.sync_copy(eb.at[w], idxv)

        @pl.loop(0, epw // LANES)
        def _(j):
            ids = idxv[pl.ds(j * LANES, LANES)]
            plsc.addupdate_scatter(hist, [ids], one16)

        pltpu.sync_copy(hist, degp_hbm.at[direction, w])


def _sc_agg(h_hbm, src_hbm, dst_hbm, aggp_hbm, idxs, idxd, rows2, shared, sem):
    np_, d = shared.shape
    seg = np_ // NS
    c = lax.axis_index("c")
    s = lax.axis_index("s")
    w = c * NS + s
    ch = idxs.shape[1]
    zero16 = jnp.zeros((LANES,), jnp.float32)

    @pl.loop(0, ch)
    def _(j):
        for k in range(d // LANES):
            rows2[0, j, pl.ds(k * LANES, LANES)] = zero16

    for k in range(seg // ch):
        pltpu.sync_copy(rows2.at[0], shared.at[pl.ds(s * seg + k * ch, ch)])
    plsc.subcore_barrier()

    nch = src_hbm.shape[1]
    bch = idxs.shape[0]

    @pl.loop(0, nch // bch)
    def _(b):
        pltpu.sync_copy(src_hbm.at[w, pl.ds(b * bch, bch)], idxs)
        pltpu.sync_copy(dst_hbm.at[w, pl.ds(b * bch, bch)], idxd)
        pltpu.async_copy(h_hbm.at[idxs.at[0]], rows2.at[0], sem.at[0])

        @pl.loop(0, bch)
        def _(j):
            p = lax.rem(j, 2)
            q = 1 - p

            @pl.when(j + 1 < bch)
            def _():
                pltpu.async_copy(h_hbm.at[idxs.at[j + 1]], rows2.at[q],
                                 sem.at[q])

            pltpu.make_async_copy(h_hbm.at[idxs.at[j]], rows2.at[p],
                                  sem.at[p]).wait()
            pltpu.sync_copy(rows2.at[p], shared.at[idxd.at[j]], add=True)

    plsc.subcore_barrier()
    pltpu.sync_copy(shared.at[pl.ds(s * seg, seg)],
                    aggp_hbm.at[c, pl.ds(s * seg, seg)])


def _tc_norm_body(n, np_, x_ref, degp_ref, h_ref, nd_ref):
    do = jnp.sum(degp_ref[0], axis=0)
    di = jnp.sum(degp_ref[1], axis=0)
    ns_ = lax.rsqrt(jnp.maximum(do, 1.0))
    nd_ref[...] = lax.rsqrt(jnp.maximum(di, 1.0)).reshape(1, np_)
    h_ref[pl.ds(0, n), :] = x_ref[...] * ns_[:n][:, None]


def _tc_main_body(n, s_seq, l_seq, d,
                  aggp_ref, nd_ref, seq_ref, wc_ref, bc_ref, w1_ref, b1_ref,
                  wih_ref, whh_ref, bi_ref, bh_ref, w2_ref, b2_ref,
                  final_ref, outv, xv, xgv, btv):
    agg = aggp_ref[0, pl.ds(0, n), :] + aggp_ref[1, pl.ds(0, n), :]
    agg = agg * nd_ref[0, :n][:, None]
    med = jnp.dot(agg, wc_ref[...], preferred_element_type=jnp.float32) + bc_ref[...]
    outv[...] = jnp.dot(med, w1_ref[...], preferred_element_type=jnp.float32) + b1_ref[...]
    btv[...] = whh_ref[...].T
    bt = btv[...]
    bsum = bi_ref[...] + bh_ref[...]
    dn_t = (((1,), (1,)), ((), ()))

    def seq_body(si, _):
        def gather_body(tb, _):
            for u in range(4):
                t = tb * 4 + u
                idx = seq_ref[si, t]
                xv[pl.ds(t, 1), :] = outv[pl.ds(idx, 1), :]
            return 0
        lax.fori_loop(0, l_seq // 4, gather_body, 0)
        xgv[...] = lax.dot_general(
            xv[...], wih_ref[...], dn_t,
            preferred_element_type=jnp.float32) + bsum

        def step_block(tb, hc):
            h, cc = hc
            for u in range(8):
                t = tb * 8 + u
                hc = h.reshape(d, 1)
                g = xgv[pl.ds(t, 1), :] + jnp.sum(
                    hc * bt, axis=0, keepdims=True)
                ig = 0.5 + 0.5 * jnp.tanh(0.5 * g[:, 0:d])
                fg = 0.5 + 0.5 * jnp.tanh(0.5 * g[:, d:2 * d])
                gg = jnp.tanh(g[:, 2 * d:3 * d])
                og = 0.5 + 0.5 * jnp.tanh(0.5 * g[:, 3 * d:4 * d])
                cc = fg * cc + ig * gg
                h = og * jnp.tanh(cc)
                oidx = seq_ref[si, t]
                outv[pl.ds(oidx, 1), :] = h
            return (h, cc)

        zero_h = jnp.zeros((1, d), jnp.float32)
        lax.fori_loop(0, l_seq // 8, step_block, (zero_h, zero_h))
        return 0

    lax.fori_loop(0, s_seq, seq_body, 0)
    final_ref[...] = jnp.dot(outv[...], w2_ref[...],
                             preferred_element_type=jnp.float32) + b2_ref[...]


def kernel(in_feats, edge_index, seq_ids, W_conv, b_conv, W_ff1, b_ff1,
           W_ih, W_hh, b_ih, b_hh, W_ff2, b_ff2):
    n, d = in_feats.shape
    e = edge_index.shape[1]
    s_seq, l_seq = seq_ids.shape
    np_ = ((n + NS * LANES - 1) // (NS * LANES)) * (NS * LANES)
    epw = e // NW
    nch = epw // CH

    src2 = edge_index[0].reshape(NW, epw)
    dst2 = edge_index[1].reshape(NW, epw)
    epw_p = ((epw + 8 * CH - 1) // (8 * CH)) * (8 * CH)
    npad = epw_p - epw
    nchp = epw_p // CH
    padb = jnp.broadcast_to(
        n + (jnp.arange(npad, dtype=jnp.int32) % (np_ - n)), (NW, npad))
    src3 = jnp.concatenate([src2, padb], axis=1).reshape(NW, nchp, CH)
    dst3 = jnp.concatenate([dst2, padb], axis=1).reshape(NW, nchp, CH)

    mesh = plsc.VectorSubcoreMesh(core_axis_name="c", subcore_axis_name="s")
    seg = np_ // NS

    degp = pl.kernel(
        _sc_degrees,
        out_type=jax.ShapeDtypeStruct((2, NW, np_), jnp.float32),
        mesh=mesh,
        compiler_params=pltpu.CompilerParams(needs_layout_passes=False),
        scratch_types=[
            pltpu.VMEM((epw,), jnp.int32),
            pltpu.VMEM((np_,), jnp.float32),
        ],
    )(src2, dst2)

    h, norm_dst = pl.pallas_call(
        functools.partial(_tc_norm_body, n, np_),
        out_shape=(
            jax.ShapeDtypeStruct((np_, d), jnp.float32),
            jax.ShapeDtypeStruct((1, np_), jnp.float32),
        ),
    )(in_feats, degp)

    aggp = pl.kernel(
        _sc_agg,
        out_type=jax.ShapeDtypeStruct((2, np_, d), jnp.float32),
        mesh=mesh,
        scratch_types=[
            pltpu.VMEM((16, CH), jnp.int32),
            pltpu.VMEM((16, CH), jnp.int32),
            pltpu.VMEM((2, CH, d), jnp.float32),
            pltpu.VMEM_SHARED((np_, d), jnp.float32),
            pltpu.SemaphoreType.DMA((2,)),
        ],
    )(h, src3, dst3)

    final = pl.pallas_call(
        functools.partial(_tc_main_body, n, s_seq, l_seq, d),
        out_shape=jax.ShapeDtypeStruct((n, d), jnp.float32),
        in_specs=[
            pl.BlockSpec(memory_space=pltpu.VMEM),
            pl.BlockSpec(memory_space=pltpu.VMEM),
            pl.BlockSpec(memory_space=pltpu.SMEM),
        ] + [pl.BlockSpec(memory_space=pltpu.VMEM)] * 10,
        out_specs=pl.BlockSpec(memory_space=pltpu.VMEM),
        scratch_shapes=[
            pltpu.VMEM((n, d), jnp.float32),
            pltpu.VMEM((l_seq, d), jnp.float32),
            pltpu.VMEM((l_seq, 4 * d), jnp.float32),
            pltpu.VMEM((d, 4 * d), jnp.float32),
        ],
    )(aggp, norm_dst, seq_ids,
      W_conv, b_conv.reshape(1, d), W_ff1, b_ff1.reshape(1, d),
      W_ih, W_hh, b_ih.reshape(1, 4 * d), b_hh.reshape(1, 4 * d),
      W_ff2, b_ff2.reshape(1, d))
    return final


# speedup vs baseline: 1.5798x; 1.0003x over previous
import functools

import jax
import jax.numpy as jnp
from jax import lax
from jax.experimental import pallas as pl
from jax.experimental.pallas import tpu as pltpu
from jax.experimental.pallas import tpu_sc as plsc

NW = 32
NS = 16
LANES = 16
CH = 128


def _sc_degrees(src_hbm, dst_hbm, degp_hbm, idxv, hist):
    np_, = hist.shape
    c = lax.axis_index("c")
    s = lax.axis_index("s")
    w = c * NS + s
    epw, = idxv.shape
    zero16 = jnp.zeros((LANES,), jnp.float32)
    one16 = jnp.full((LANES,), 1.0, jnp.float32)
    for direction, eb in enumerate((src_hbm, dst_hbm)):
        @pl.loop(0, np_ // LANES)
        def _(i):
            hist[pl.ds(i * LANES, LANES)] = zero16

        pltpu.sync_copy(eb.at[w], idxv)

        @pl.loop(0, epw // LANES)
        def _(j):
            ids = idxv[pl.ds(j * LANES, LANES)]
            plsc.addupdate_scatter(hist, [ids], one16)

        pltpu.sync_copy(hist, degp_hbm.at[direction, w])


def _sc_agg(h_hbm, src_hbm, dst_hbm, aggp_hbm, idxs, idxd, rows2, shared, sem):
    np_, d = shared.shape
    seg = np_ // NS
    c = lax.axis_index("c")
    s = lax.axis_index("s")
    w = c * NS + s
    ch = idxs.shape[1]
    zero16 = jnp.zeros((LANES,), jnp.float32)

    @pl.loop(0, ch)
    def _(j):
        for k in range(d // LANES):
            rows2[0, j, pl.ds(k * LANES, LANES)] = zero16

    for k in range(seg // ch):
        pltpu.sync_copy(rows2.at[0], shared.at[pl.ds(s * seg + k * ch, ch)])
    plsc.subcore_barrier()

    nch = src_hbm.shape[1]
    bch = idxs.shape[0]

    @pl.loop(0, nch // bch)
    def _(b):
        pltpu.sync_copy(src_hbm.at[w, pl.ds(b * bch, bch)], idxs)
        pltpu.sync_copy(dst_hbm.at[w, pl.ds(b * bch, bch)], idxd)
        pltpu.async_copy(h_hbm.at[idxs.at[0]], rows2.at[0], sem.at[0])

        @pl.loop(0, bch)
        def _(j):
            p = lax.rem(j, 2)
            q = 1 - p

            @pl.when(j + 1 < bch)
            def _():
                pltpu.async_copy(h_hbm.at[idxs.at[j + 1]], rows2.at[q],
                                 sem.at[q])

            pltpu.make_async_copy(h_hbm.at[idxs.at[j]], rows2.at[p],
                                  sem.at[p]).wait()
            pltpu.sync_copy(rows2.at[p], shared.at[idxd.at[j]], add=True)

    plsc.subcore_barrier()
    pltpu.sync_copy(shared.at[pl.ds(s * seg, seg)],
                    aggp_hbm.at[c, pl.ds(s * seg, seg)])


def _tc_norm_body(n, np_, x_ref, degp_ref, h_ref, nd_ref):
    do = jnp.sum(degp_ref[0], axis=0)
    di = jnp.sum(degp_ref[1], axis=0)
    ns_ = lax.rsqrt(jnp.maximum(do, 1.0))
    nd_ref[...] = lax.rsqrt(jnp.maximum(di, 1.0)).reshape(1, np_)
    h_ref[pl.ds(0, n), :] = x_ref[...] * ns_[:n][:, None]


def _tc_main_body(n, s_seq, l_seq, d,
                  aggp_ref, nd_ref, seq_ref, wc_ref, bc_ref, w1_ref, b1_ref,
                  wih_ref, whh_ref, bi_ref, bh_ref, w2_ref, b2_ref,
                  final_ref, outv, xv, xgv, btv):
    agg = aggp_ref[0, pl.ds(0, n), :] + aggp_ref[1, pl.ds(0, n), :]
    agg = agg * nd_ref[0, :n][:, None]
    med = jnp.dot(agg, wc_ref[...], preferred_element_type=jnp.float32) + bc_ref[...]
    outv[...] = jnp.dot(med, w1_ref[...], preferred_element_type=jnp.float32) + b1_ref[...]
    btv[...] = whh_ref[...].T
    bt = btv[...]
    bsum = bi_ref[...] + bh_ref[...]
    dn_t = (((1,), (1,)), ((), ()))

    def seq_body(si, _):
        def gather_body(tb, _):
            tb4 = pl.multiple_of(tb * 4, 4)
            for u in range(4):
                t = tb4 + u
                idx = seq_ref[si, t]
                xv[pl.ds(t, 1), :] = outv[pl.ds(idx, 1), :]
            return 0
        lax.fori_loop(0, l_seq // 4, gather_body, 0)
        xgv[...] = lax.dot_general(
            xv[...], wih_ref[...], dn_t,
            preferred_element_type=jnp.float32) + bsum

        def step_block(tb, hc):
            h, cc = hc
            tb8 = pl.multiple_of(tb * 8, 8)
            for u in range(8):
                t = tb8 + u
                hc = h.reshape(d, 1)
                g = xgv[pl.ds(t, 1), :] + jnp.sum(
                    hc * bt, axis=0, keepdims=True)
                ig = 0.5 + 0.5 * jnp.tanh(0.5 * g[:, 0:d])
                fg = 0.5 + 0.5 * jnp.tanh(0.5 * g[:, d:2 * d])
                gg = jnp.tanh(g[:, 2 * d:3 * d])
                og = 0.5 + 0.5 * jnp.tanh(0.5 * g[:, 3 * d:4 * d])
                cc = fg * cc + ig * gg
                h = og * jnp.tanh(cc)
                oidx = seq_ref[si, t]
                outv[pl.ds(oidx, 1), :] = h
            return (h, cc)

        zero_h = jnp.zeros((1, d), jnp.float32)
        lax.fori_loop(0, l_seq // 8, step_block, (zero_h, zero_h))
        return 0

    lax.fori_loop(0, s_seq, seq_body, 0)
    final_ref[...] = jnp.dot(outv[...], w2_ref[...],
                             preferred_element_type=jnp.float32) + b2_ref[...]


def kernel(in_feats, edge_index, seq_ids, W_conv, b_conv, W_ff1, b_ff1,
           W_ih, W_hh, b_ih, b_hh, W_ff2, b_ff2):
    n, d = in_feats.shape
    e = edge_index.shape[1]
    s_seq, l_seq = seq_ids.shape
    np_ = ((n + NS * LANES - 1) // (NS * LANES)) * (NS * LANES)
    epw = e // NW

    src2 = edge_index[0].reshape(NW, epw)
    dst2 = edge_index[1].reshape(NW, epw)
    epw_p = ((epw + 8 * CH - 1) // (8 * CH)) * (8 * CH)
    npad = epw_p - epw
    nchp = epw_p // CH
    padb = jnp.broadcast_to(
        n + (jnp.arange(npad, dtype=jnp.int32) % (np_ - n)), (NW, npad))
    src3 = jnp.concatenate([src2, padb], axis=1).reshape(NW, nchp, CH)
    dst3 = jnp.concatenate([dst2, padb], axis=1).reshape(NW, nchp, CH)

    mesh = plsc.VectorSubcoreMesh(core_axis_name="c", subcore_axis_name="s")

    degp = pl.kernel(
        _sc_degrees,
        out_type=jax.ShapeDtypeStruct((2, NW, np_), jnp.float32),
        mesh=mesh,
        compiler_params=pltpu.CompilerParams(needs_layout_passes=False),
        scratch_types=[
            pltpu.VMEM((epw,), jnp.int32),
            pltpu.VMEM((np_,), jnp.float32),
        ],
    )(src2, dst2)

    h, norm_dst = pl.pallas_call(
        functools.partial(_tc_norm_body, n, np_),
        out_shape=(
            jax.ShapeDtypeStruct((np_, d), jnp.float32),
            jax.ShapeDtypeStruct((1, np_), jnp.float32),
        ),
    )(in_feats, degp)

    aggp = pl.kernel(
        _sc_agg,
        out_type=jax.ShapeDtypeStruct((2, np_, d), jnp.float32),
        mesh=mesh,
        scratch_types=[
            pltpu.VMEM((16, CH), jnp.int32),
            pltpu.VMEM((16, CH), jnp.int32),
            pltpu.VMEM((2, CH, d), jnp.float32),
            pltpu.VMEM_SHARED((np_, d), jnp.float32),
            pltpu.SemaphoreType.DMA((2,)),
        ],
    )(h, src3, dst3)

    final = pl.pallas_call(
        functools.partial(_tc_main_body, n, s_seq, l_seq, d),
        out_shape=jax.ShapeDtypeStruct((n, d), jnp.float32),
        in_specs=[
            pl.BlockSpec(memory_space=pltpu.VMEM),
            pl.BlockSpec(memory_space=pltpu.VMEM),
            pl.BlockSpec(memory_space=pltpu.SMEM),
        ] + [pl.BlockSpec(memory_space=pltpu.VMEM)] * 10,
        out_specs=pl.BlockSpec(memory_space=pltpu.VMEM),
        scratch_shapes=[
            pltpu.VMEM((n, d), jnp.float32),
            pltpu.VMEM((l_seq, d), jnp.float32),
            pltpu.VMEM((l_seq, 4 * d), jnp.float32),
            pltpu.VMEM((d, 4 * d), jnp.float32),
        ],
    )(aggp, norm_dst, seq_ids,
      W_conv, b_conv.reshape(1, d), W_ff1, b_ff1.reshape(1, d),
      W_ih, W_hh, b_ih.reshape(1, 4 * d), b_hh.reshape(1, 4 * d),
      W_ff2, b_ff2.reshape(1, d))
    return final
